# baseline (device time: 48068 ns/iter reference)
import jax
import jax.numpy as jnp
from jax import lax
from jax.experimental import pallas as pl
from jax.experimental.pallas import tpu as pltpu

N_DEV = 8

_A, _B = slice(0, 2), slice(2, 4)
_SENDS = {
    0: (0, _A, 1, 1), 1: (0, _B, 1, 1),
    2: (0, _A, 3, 3), 3: (0, _B, 3, 3),
    4: (0, _A, 4, 4), 5: (0, _B, 4, 4),
    6: (1, _A, 2, 3), 7: (1, _B, 2, 3),
    8: (3, _A, 7, 4), 9: (3, _B, 7, 4),
    10: (4, _A, 5, 1), 11: (4, _B, 5, 1),
    12: (7, _A, 6, 1),
    13: (5, _B, 6, 3),
}
N_SEMS = len(_SENDS)


def kernel(x, router_W, route_idx, expert_W, shared_W):
    n_tok, d = x.shape
    e_loc, _, h = expert_W.shape

    def body(x_ref, rW_ref, idx_ref, eW_ref, sW_ref, out_ref,
             chunks_ref, send_sems, recv_sems):
        my = lax.axis_index("i")
        nbr = {m: jnp.bitwise_xor(my, m) for m in (1, 3, 4)}

        chunks_ref[0] = eW_ref[...].astype(jnp.bfloat16)

        barrier_sem = pltpu.get_barrier_semaphore()
        for m in (1, 3, 4):
            pl.semaphore_signal(
                barrier_sem, inc=1,
                device_id=(nbr[m],), device_id_type=pl.DeviceIdType.MESH,
            )
        pl.semaphore_wait(barrier_sem, 3)

        def make_send(k):
            src, es, dst, m = _SENDS[k]
            return pltpu.make_async_remote_copy(
                src_ref=chunks_ref.at[src, es],
                dst_ref=chunks_ref.at[dst, es],
                send_sem=send_sems.at[k],
                recv_sem=recv_sems.at[k],
                device_id=(nbr[m],),
                device_id_type=pl.DeviceIdType.MESH,
            )

        rdmas = {}
        for k in range(6):
            rdmas[k] = make_send(k)
            rdmas[k].start()

        xv = x_ref[...]
        xb = xv.astype(jnp.bfloat16)
        idx = idx_ref[...]

        scores = jnp.dot(xv, rW_ref[...], preferred_element_type=jnp.float32)
        smax = jnp.max(scores, axis=1, keepdims=True)
        probs = jnp.exp(scores - smax)
        probs = probs / jnp.sum(probs, axis=1, keepdims=True)
        eids = lax.broadcasted_iota(jnp.int32, scores.shape, 1)
        p = jnp.sum(jnp.where(eids == idx, probs, 0.0), axis=1, keepdims=True)

        acc = jnp.dot(xv, sW_ref[...], preferred_element_type=jnp.float32)

        def add_chunk(acc, slot):
            origin = jnp.bitwise_xor(my, slot)
            for j in range(e_loc):
                e = origin * e_loc + j
                gate = (p * (idx == e).astype(jnp.float32)).astype(jnp.bfloat16)
                acc = acc + jnp.dot(
                    xb * gate, chunks_ref[slot, j],
                    preferred_element_type=jnp.float32,
                )
            return acc

        acc = add_chunk(acc, 0)

        def step(k, fwd=None):
            rdmas[k].wait()
            if fwd is not None:
                rdmas[fwd] = make_send(fwd)
                rdmas[fwd].start()

        step(0, 6); step(1, 7);   acc = add_chunk(acc, 1)
        step(2, 8); step(3, 9);   acc = add_chunk(acc, 3)
        step(4, 10); step(5, 11); acc = add_chunk(acc, 4)
        step(8, 12); step(9);     acc = add_chunk(acc, 7)
        step(11, 13); step(10);   acc = add_chunk(acc, 5)
        step(6); step(7);         acc = add_chunk(acc, 2)
        step(12); step(13);       acc = add_chunk(acc, 6)

        out_ref[...] = acc

    return pl.pallas_call(
        body,
        out_shape=jax.ShapeDtypeStruct((n_tok, h), jnp.float32),
        in_specs=[pl.BlockSpec(memory_space=pltpu.VMEM)] * 5,
        out_specs=pl.BlockSpec(memory_space=pltpu.VMEM),
        scratch_shapes=[
            pltpu.VMEM((N_DEV, e_loc, d, h), jnp.bfloat16),
            pltpu.SemaphoreType.DMA((N_SEMS,)),
            pltpu.SemaphoreType.DMA((N_SEMS,)),
        ],
        compiler_params=pltpu.CompilerParams(collective_id=0),
    )(x, router_W, route_idx, expert_W, shared_W)
